# baseline (speedup 1.0000x reference)
import functools

import jax
import jax.numpy as jnp
from jax import lax
from jax.experimental import pallas as pl
from jax.experimental.pallas import tpu as pltpu
from jax.experimental.pallas import tpu_sc as plsc

N_ROWS = 1_000_000
EMB_DIM = 32
BATCH = 16384

_info = plsc.get_sparse_core_info()
_NC, _NS = _info.num_cores, _info.num_subcores
_NW = _NC * _NS
_BPW = BATCH // _NW


def _build():
    mesh = plsc.VectorSubcoreMesh(core_axis_name="c", subcore_axis_name="s")
    out_sds = jax.ShapeDtypeStruct((BATCH, EMB_DIM), jnp.float32)

    @functools.partial(
        pl.kernel,
        out_type=(out_sds, out_sds, out_sds),
        mesh=mesh,
        compiler_params=pltpu.CompilerParams(use_tc_tiling_on_sc=False),
        scratch_types=[
            pltpu.VMEM((_BPW,), jnp.int32),
            pltpu.VMEM((_BPW,), jnp.int32),
            pltpu.VMEM((_BPW,), jnp.int32),
            pltpu.VMEM((_BPW, EMB_DIM), jnp.float32),
            pltpu.VMEM((_BPW, EMB_DIM), jnp.float32),
            pltpu.VMEM((_BPW, EMB_DIM), jnp.float32),
            pltpu.SemaphoreType.DMA,
            pltpu.SemaphoreType.DMA,
        ],
    )
    def gather3(table, u_hbm, p_hbm, n_hbm, out_u, out_p, out_n,
                i0, i1, i2, r0, r1, r2, sem_g, sem_s):
        wid = lax.axis_index("s") * _NC + lax.axis_index("c")
        base = wid * _BPW
        idx_refs = (i0, i1, i2)
        row_refs = (r0, r1, r2)
        in_refs = (u_hbm, p_hbm, n_hbm)
        out_refs = (out_u, out_p, out_n)

        for j in range(3):
            pltpu.sync_copy(in_refs[j].at[pl.ds(base, _BPW)], idx_refs[j])
        gathers = [
            pltpu.async_copy(table.at[idx_refs[j]], row_refs[j], sem_g)
            for j in range(3)
        ]
        stores = []
        for j in range(3):
            gathers[j].wait()
            stores.append(
                pltpu.async_copy(
                    row_refs[j], out_refs[j].at[pl.ds(base, _BPW)], sem_s
                )
            )
        for s in stores:
            s.wait()

    return gather3


_gather3 = _build()


def kernel(embeds, users, pos_items, neg_items):
    u, p, n = _gather3(embeds, users, pos_items, neg_items)
    return (u, p, n, u, p, n)
